# baseline (speedup 1.0000x reference)
import jax
import jax.numpy as jnp
from jax.experimental import pallas as pl

_NC = 1000
_W = 1024
_R = 1024


def _onehot_flat_block(a1_ref, a2_ref, a3_ref, o_ref):
    li = jax.lax.broadcasted_iota(jnp.int32, o_ref.shape, 1)
    hot = (li == a1_ref[...]) | (li == a2_ref[...]) | (li == a3_ref[...])
    o_ref[...] = hot.astype(jnp.float32)


def kernel(x):
    B, S = x.shape
    n = B * S
    nflat = n * _NC // _W

    xf = x.reshape(n).astype(jnp.int32)
    p = jnp.arange(n, dtype=jnp.int32) * _NC + xf
    t = jnp.arange(nflat, dtype=jnp.int32)
    base = t * _W
    r1 = base // _NC
    p1 = jnp.take(p, r1, fill_value=-1)
    p2 = jnp.take(p, r1 + 1, fill_value=-1)
    p3 = jnp.take(p, r1 + 2, fill_value=-1)
    a1 = jnp.where((p1 >= base) & (p1 < base + _W), p1 - base, -1)
    a2 = jnp.where((p2 >= base) & (p2 < base + _W), p2 - base, -1)
    a3 = jnp.where((p3 >= base) & (p3 < base + _W), p3 - base, -1)

    y = pl.pallas_call(
        _onehot_flat_block,
        grid=(nflat // _R,),
        in_specs=[
            pl.BlockSpec((_R, 1), lambda i: (i, 0)),
            pl.BlockSpec((_R, 1), lambda i: (i, 0)),
            pl.BlockSpec((_R, 1), lambda i: (i, 0)),
        ],
        out_specs=pl.BlockSpec((_R, _W), lambda i: (i, 0)),
        out_shape=jax.ShapeDtypeStruct((nflat, _W), jnp.float32),
    )(a1.reshape(nflat, 1), a2.reshape(nflat, 1), a3.reshape(nflat, 1))
    return y.reshape(B, S, _NC)


# speedup vs baseline: 1.0016x; 1.0016x over previous
import jax
import jax.numpy as jnp
from jax.experimental import pallas as pl

_NC = 1000
_W = 1024
_R = 1040


def _onehot_flat_block(a1_ref, a2_ref, a3_ref, o_ref):
    li = jax.lax.broadcasted_iota(jnp.int32, o_ref.shape, 1)
    hot = (li == a1_ref[...]) | (li == a2_ref[...]) | (li == a3_ref[...])
    o_ref[...] = hot.astype(jnp.float32)


def kernel(x):
    B, S = x.shape
    n = B * S
    nflat = n * _NC // _W

    xf = x.reshape(n).astype(jnp.int32)
    p = jnp.arange(n, dtype=jnp.int32) * _NC + xf
    t = jnp.arange(nflat, dtype=jnp.int32)
    base = t * _W
    r1 = base // _NC
    p1 = jnp.take(p, r1, fill_value=-1)
    p2 = jnp.take(p, r1 + 1, fill_value=-1)
    p3 = jnp.take(p, r1 + 2, fill_value=-1)
    a1 = jnp.where((p1 >= base) & (p1 < base + _W), p1 - base, -1)
    a2 = jnp.where((p2 >= base) & (p2 < base + _W), p2 - base, -1)
    a3 = jnp.where((p3 >= base) & (p3 < base + _W), p3 - base, -1)

    y = pl.pallas_call(
        _onehot_flat_block,
        grid=(nflat // _R,),
        in_specs=[
            pl.BlockSpec((_R, 1), lambda i: (i, 0)),
            pl.BlockSpec((_R, 1), lambda i: (i, 0)),
            pl.BlockSpec((_R, 1), lambda i: (i, 0)),
        ],
        out_specs=pl.BlockSpec((_R, _W), lambda i: (i, 0)),
        out_shape=jax.ShapeDtypeStruct((nflat, _W), jnp.float32),
    )(a1.reshape(nflat, 1), a2.reshape(nflat, 1), a3.reshape(nflat, 1))
    return y.reshape(B, S, _NC)


# speedup vs baseline: 1.4465x; 1.4442x over previous
import jax
import jax.numpy as jnp
from jax.experimental import pallas as pl

_NC = 1000
_R = 8


def _onehot_t_block(x_ref, o_ref):
    xv = x_ref[...]
    iota = jax.lax.broadcasted_iota(jnp.int32, o_ref.shape, 1)
    o_ref[...] = (xv[:, None, :] == iota).astype(jnp.float32)


def kernel(x):
    B, S = x.shape
    yt = pl.pallas_call(
        _onehot_t_block,
        grid=(B // _R,),
        in_specs=[pl.BlockSpec((_R, S), lambda i: (i, 0))],
        out_specs=pl.BlockSpec((_R, _NC, S), lambda i: (i, 0, 0)),
        out_shape=jax.ShapeDtypeStruct((B, _NC, S), jnp.float32),
    )(x)
    return yt.transpose(0, 2, 1)


# speedup vs baseline: 14.6644x; 10.1381x over previous
import jax
import jax.numpy as jnp
from jax.experimental import pallas as pl

_NC = 1000


def _onehot_plane(xt_ref, o_ref):
    xv = xt_ref[...]
    iota = jax.lax.broadcasted_iota(jnp.int32, o_ref.shape, 1)
    o_ref[...] = (xv == iota).astype(jnp.float32)


def kernel(x):
    B, S = x.shape
    xt = x.T.reshape(S, 1, B)
    yt = pl.pallas_call(
        _onehot_plane,
        grid=(S,),
        in_specs=[pl.BlockSpec((1, 1, B), lambda j: (j, 0, 0))],
        out_specs=pl.BlockSpec((1, _NC, B), lambda j: (j, 0, 0)),
        out_shape=jax.ShapeDtypeStruct((S, _NC, B), jnp.float32),
    )(xt)
    return yt.transpose(2, 0, 1)


# speedup vs baseline: 14.7429x; 1.0054x over previous
import jax
import jax.numpy as jnp
from jax.experimental import pallas as pl
from jax.experimental.pallas import tpu as pltpu

_NC = 1000
_BC = 1024
_NBUF = 8


def _onehot_ring(xt_ref, o_ref, buf, sem):
    i = pl.program_id(0)
    nsteps = pl.num_programs(0)
    nchunk = o_ref.shape[2] // _BC
    j = i // nchunk
    bb = jax.lax.rem(i, nchunk)
    slot = jax.lax.rem(i, _NBUF)

    @pl.when(i >= _NBUF)
    def _():
        pltpu.make_async_copy(
            buf.at[slot], o_ref.at[0, :, pl.ds(0, _BC)], sem.at[slot]
        ).wait()

    iota = jax.lax.broadcasted_iota(jnp.int32, (_NC, _BC), 0)
    buf[slot] = (xt_ref[0] == iota).astype(jnp.float32)
    pltpu.make_async_copy(
        buf.at[slot], o_ref.at[j, :, pl.ds(bb * _BC, _BC)], sem.at[slot]
    ).start()

    @pl.when(i == nsteps - 1)
    def _():
        for k in range(_NBUF):
            pltpu.make_async_copy(
                buf.at[k], o_ref.at[0, :, pl.ds(0, _BC)], sem.at[k]
            ).wait()


def kernel(x):
    B, S = x.shape
    nchunk = B // _BC
    xt = x.T.reshape(S * nchunk, 1, _BC)
    yt = pl.pallas_call(
        _onehot_ring,
        grid=(S * nchunk,),
        in_specs=[pl.BlockSpec((1, 1, _BC), lambda i: (i, 0, 0))],
        out_specs=pl.BlockSpec(memory_space=pl.ANY),
        out_shape=jax.ShapeDtypeStruct((S, _NC, B), jnp.float32),
        scratch_shapes=[
            pltpu.VMEM((_NBUF, _NC, _BC), jnp.float32),
            pltpu.SemaphoreType.DMA((_NBUF,)),
        ],
    )(xt)
    return yt.transpose(2, 0, 1)
